# baseline (speedup 1.0000x reference)
import functools

import jax
import jax.numpy as jnp
from jax import lax
from jax.experimental import pallas as pl
from jax.experimental.pallas import tpu as pltpu
from jax.experimental.pallas import tpu_sc as plsc

N = 16384
SLOTS = 4
P = 33
DELTA = 1.0 / 4200.0 * 5.0
STEP = 2.0 * DELTA / (SLOTS - 1)

NC = 2
NS = 16
L = 16
NW = NC * NS
XC = N // NW
OC = XC * SLOTS
NVEC = OC // L
YPAD = 40


def _sc_body(x_hbm, y_hbm, out_hbm, x_v, y_v, out_v):
    wid = lax.axis_index("s") * NC + lax.axis_index("c")

    pltpu.sync_copy(x_hbm.at[pl.ds(wid * XC, XC)], x_v)
    pltpu.sync_copy(y_hbm, y_v)

    iota = lax.iota(jnp.int32, L)
    x_sel = lax.shift_right_logical(iota, 2)
    zeros = iota & 0
    off_v = (iota & 3).astype(jnp.float32) * STEP - DELTA

    def body(j, _):
        ix = x_sel + j * (L // SLOTS)
        xv = plsc.load_gather(x_v, [ix])
        t = (xv + off_v) * 32.0
        k = t.astype(jnp.int32)
        k = jnp.minimum(jnp.maximum(k, 0), P - 2)
        y1 = plsc.load_gather(y_v, [zeros, k])
        y2 = plsc.load_gather(y_v, [zeros, k + 1])
        w = t - k.astype(jnp.float32)
        out_v[j >> 3, pl.ds((j & 7) * L, L)] = y1 + w * (y2 - y1)
        return _

    lax.fori_loop(0, NVEC, body, 0, unroll=8)

    pltpu.sync_copy(out_v, out_hbm.at[wid])


@jax.jit
def kernel(x, y_points):
    mesh = plsc.VectorSubcoreMesh(core_axis_name="c", subcore_axis_name="s")
    run = pl.kernel(
        _sc_body,
        out_type=jax.ShapeDtypeStruct((NW, OC // 128, 128), jnp.float32),
        mesh=mesh,
        scratch_types=[
            pltpu.VMEM((XC,), jnp.float32),
            pltpu.VMEM((1, P), jnp.float32),
            pltpu.VMEM((OC // 128, 128), jnp.float32),
        ],
        compiler_params=pltpu.CompilerParams(needs_layout_passes=False),
    )
    return run(x, y_points).reshape(N, SLOTS)


# speedup vs baseline: 1.7954x; 1.7954x over previous
import jax
import jax.numpy as jnp
from jax import lax
from jax.experimental import pallas as pl
from jax.experimental.pallas import tpu as pltpu
from jax.experimental.pallas import tpu_sc as plsc

N = 16384
SLOTS = 4
P = 33
DELTA = 1.0 / 4200.0 * 5.0
STEP = 2.0 * DELTA / (SLOTS - 1)

NC = 2
NS = 16
L = 16
NW = NC * NS
XC = N // NW
ROWS = XC * SLOTS // 128
NCHUNK = XC // L


def _sc_body(x_hbm, y_hbm, out_hbm, x_v, y_v, out_v):
    wid = lax.axis_index("s") * NC + lax.axis_index("c")

    pltpu.sync_copy(x_hbm.at[pl.ds(wid * XC, XC)], x_v)
    pltpu.sync_copy(y_hbm, y_v)

    iota = lax.iota(jnp.int32, L)
    zeros = iota & 0

    def body(j, _):
        trow = lax.shift_right_logical(j, 3) * SLOTS
        col = (j & 7) * L
        xv = x_v[pl.ds(j * L, L)]
        t0 = xv * 32.0
        for s in range(SLOTS):
            t = t0 + (s * STEP - DELTA) * 32.0
            k = t.astype(jnp.int32)
            k = jnp.minimum(jnp.maximum(k, 0), P - 2)
            y1 = plsc.load_gather(y_v, [zeros, k])
            y2 = plsc.load_gather(y_v, [zeros, k + 1])
            w = t - k.astype(jnp.float32)
            out_v[trow + s, pl.ds(col, L)] = y1 + w * (y2 - y1)
        return _

    lax.fori_loop(0, NCHUNK, body, 0, unroll=4)

    pltpu.sync_copy(out_v, out_hbm.at[pl.ds(wid * ROWS, ROWS)])


@jax.jit
def kernel(x, y_points):
    mesh = plsc.VectorSubcoreMesh(core_axis_name="c", subcore_axis_name="s")
    run = pl.kernel(
        _sc_body,
        out_type=jax.ShapeDtypeStruct((N * SLOTS // 128, 128), jnp.float32),
        mesh=mesh,
        scratch_types=[
            pltpu.VMEM((XC,), jnp.float32),
            pltpu.VMEM((1, P), jnp.float32),
            pltpu.VMEM((ROWS, 128), jnp.float32),
        ],
        compiler_params=pltpu.CompilerParams(needs_layout_passes=False),
    )
    r = run(x, y_points)
    return r.reshape(N // 128, SLOTS, 128).swapaxes(1, 2).reshape(N, SLOTS)
